# Initial kernel scaffold; baseline (speedup 1.0000x reference)
import jax
import jax.numpy as jnp
from jax.experimental import pallas as pl


def kernel(x, y):
    raise NotImplementedError("write your pallas kernel here")


# speedup vs baseline: 123.5958x; 123.5958x over previous
import jax
import jax.numpy as jnp
from jax.experimental import pallas as pl
from jax.experimental.pallas import tpu as pltpu

N = 16384
DIM = 128
NUM_CLASSES = 100
CP = 104
K = 64
ALPHA = 1.0
EPS = 1e-6
NITER = 42
BIG = 1e30

_HI = jax.lax.Precision.HIGHEST
_DN = (((1,), (1,)), ((), ()))


def _loss_body(xt_ref, yrow_ref, out_ref):
    xt = xt_ref[...]
    yrow = yrow_ref[...]

    member = (jax.lax.broadcasted_iota(jnp.int32, (CP, N), 0) == yrow)
    onehot = member.astype(jnp.float32)
    onehot_b = member.astype(jnp.bfloat16)

    counts = jnp.sum(onehot, axis=1, keepdims=True)
    ones_row = jnp.ones((1, N), dtype=jnp.bfloat16)
    counts_row = jax.lax.dot_general(
        ones_row, onehot_b, _DN,
        preferred_element_type=jnp.float32)

    sums = jax.lax.dot_general(onehot, xt, _DN, precision=_HI)
    cpm = sums / jnp.maximum(counts, 1.0) + EPS

    g = jax.lax.dot(cpm, xt, precision=_HI)
    cpsq = jnp.sum(cpm * cpm, axis=1, keepdims=True)
    xsq = jnp.sum(xt * xt, axis=0, keepdims=True)
    dist = jnp.sqrt(jnp.maximum(cpsq - 2.0 * g + xsq, 0.0))

    dmax = jnp.max(dist)
    kf = jnp.float32(K)

    def neg_step(_, carry):
        lo, hi = carry
        mid = 0.5 * (lo + hi)
        t_s = jax.lax.dot(mid, onehot, precision=_HI)
        cmp = (dist <= t_s).astype(jnp.bfloat16)
        cnt = jax.lax.dot_general(
            cmp, onehot_b, _DN,
            preferred_element_type=jnp.float32)
        pred = cnt >= kf
        return (jnp.where(pred, lo, mid), jnp.where(pred, mid, hi))

    lo0 = jnp.full((CP, CP), -1.0, dtype=jnp.float32)
    hi0 = jnp.full((CP, CP), 1.0, dtype=jnp.float32) * (dmax + 1.0)
    _, v_pair = jax.lax.fori_loop(0, NITER, neg_step, (lo0, hi0))

    v_s = jax.lax.dot(v_pair, onehot, precision=_HI)
    less = dist < v_s
    cnt_l = jax.lax.dot_general(
        less.astype(jnp.bfloat16), onehot_b, _DN,
        preferred_element_type=jnp.float32)
    sum_l = jax.lax.dot_general(jnp.where(less, dist, 0.0), onehot, _DN,
                                precision=_HI)
    neg_mean = (sum_l + (kf - cnt_l) * v_pair) / kf
    neg_mean = jnp.where(counts_row < kf, BIG, neg_mean)

    def pos_step(_, carry):
        lo, hi = carry
        mid = 0.5 * (lo + hi)
        cmp = jnp.where(dist >= -mid, onehot, 0.0)
        cnt = jnp.sum(cmp, axis=1, keepdims=True)
        pred = cnt >= kf
        return (jnp.where(pred, lo, mid), jnp.where(pred, mid, hi))

    lo0p = jnp.full((CP, 1), -1.0, dtype=jnp.float32) * (dmax + 2.0)
    hi0p = jnp.zeros((CP, 1), dtype=jnp.float32)
    _, v_p = jax.lax.fori_loop(0, NITER, pos_step, (lo0p, hi0p))

    cmp_p = jnp.where(-dist < v_p, onehot, 0.0)
    cnt_lp = jnp.sum(cmp_p, axis=1, keepdims=True)
    sum_lp = jnp.sum(cmp_p * (-dist), axis=1, keepdims=True)
    pos_mean = -((sum_lp + (kf - cnt_lp) * v_p) / kf)
    pos_mean = jnp.where(counts < kf, -BIG, pos_mean)

    terms = jnp.maximum(ALPHA + pos_mean - neg_mean, 0.0)
    present = counts > 0.0
    present_row = counts_row > 0.0
    diag = (jax.lax.broadcasted_iota(jnp.int32, (CP, CP), 0)
            == jax.lax.broadcasted_iota(jnp.int32, (CP, CP), 1))
    mask = present & present_row & (~diag)
    loss = jnp.sum(jnp.where(mask, terms, 0.0))
    c_n = jnp.sum(present.astype(jnp.float32))
    loss = loss / ((c_n + 1.0) * (c_n * 0.5))
    out_ref[...] = jnp.broadcast_to(loss, (1, 1))


def kernel(x, y):
    xt = x.T
    yrow = y.astype(jnp.int32).reshape(1, N)
    out = pl.pallas_call(
        _loss_body,
        out_shape=jax.ShapeDtypeStruct((1, 1), jnp.float32),
        compiler_params=pltpu.CompilerParams(
            vmem_limit_bytes=64 * 1024 * 1024),
    )(xt, yrow)
    return out.reshape(1)


# speedup vs baseline: 126.2272x; 1.0213x over previous
import functools

import jax
import jax.numpy as jnp
from jax.experimental import pallas as pl
from jax.experimental.pallas import tpu as pltpu
from jax.experimental.pallas import tpu_sc as plsc

N = 16384
DIM = 128
NUM_CLASSES = 100
CP = 104
K = 64
ALPHA = 1.0
EPS = 1e-6
NITER = 42
BIG = 1e30

_HI = jax.lax.Precision.HIGHEST
_DN = (((1,), (1,)), ((), ()))


_SC_MESH = plsc.VectorSubcoreMesh(core_axis_name="c", subcore_axis_name="s")


@functools.partial(
    pl.kernel,
    mesh=_SC_MESH,
    out_type=jax.ShapeDtypeStruct((2, CP, DIM), jnp.float32),
    scratch_types=[
        pltpu.VMEM((128, DIM), jnp.float32),
        pltpu.VMEM((4, 128), jnp.int32),
        pltpu.VMEM_SHARED((CP, DIM), jnp.float32),
    ],
)
def _sc_center_sums(x_hbm, y_hbm, z_hbm, out_hbm, xbuf, ybuf, acc):
    cid = jax.lax.axis_index("c")
    sid = jax.lax.axis_index("s")

    @pl.when(sid == 0)
    def _():
        pltpu.sync_copy(z_hbm, acc)
    plsc.subcore_barrier()

    base = cid * 64 + sid * 4
    pltpu.sync_copy(y_hbm.at[pl.ds(base, 4)], ybuf)
    for g in range(4):
        pltpu.sync_copy(x_hbm.at[pl.ds((base + g) * 128, 128)], xbuf)
        pltpu.sync_copy(xbuf, acc.at[ybuf.at[g]], add=True)
    plsc.subcore_barrier()

    @pl.when(sid == 0)
    def _():
        pltpu.sync_copy(acc, out_hbm.at[cid])


def _loss_body(xt_ref, yrow_ref, part_ref, out_ref):
    xt = xt_ref[...]
    yrow = yrow_ref[...]

    member = (jax.lax.broadcasted_iota(jnp.int32, (CP, N), 0) == yrow)
    onehot = member.astype(jnp.float32)
    onehot_b = member.astype(jnp.bfloat16)

    counts = jnp.sum(onehot, axis=1, keepdims=True)
    ones_row = jnp.ones((1, N), dtype=jnp.bfloat16)
    counts_row = jax.lax.dot_general(
        ones_row, onehot_b, _DN,
        preferred_element_type=jnp.float32)

    part = part_ref[...]
    sums = part[0:CP, :] + part[CP:2 * CP, :]
    cpm = sums / jnp.maximum(counts, 1.0) + EPS

    g = jax.lax.dot(cpm, xt, precision=_HI)
    cpsq = jnp.sum(cpm * cpm, axis=1, keepdims=True)
    xsq = jnp.sum(xt * xt, axis=0, keepdims=True)
    dist = jnp.sqrt(jnp.maximum(cpsq - 2.0 * g + xsq, 0.0))

    dmax = jnp.max(dist)
    kf = jnp.float32(K)

    def neg_step(_, carry):
        lo, hi = carry
        mid = 0.5 * (lo + hi)
        t_s = jax.lax.dot(mid, onehot, precision=_HI)
        cmp = (dist <= t_s).astype(jnp.bfloat16)
        cnt = jax.lax.dot_general(
            cmp, onehot_b, _DN,
            preferred_element_type=jnp.float32)
        pred = cnt >= kf
        return (jnp.where(pred, lo, mid), jnp.where(pred, mid, hi))

    lo0 = jnp.full((CP, CP), -1.0, dtype=jnp.float32)
    hi0 = jnp.full((CP, CP), 1.0, dtype=jnp.float32) * (dmax + 1.0)
    _, v_pair = jax.lax.fori_loop(0, NITER, neg_step, (lo0, hi0))

    v_s = jax.lax.dot(v_pair, onehot, precision=_HI)
    less = dist < v_s
    cnt_l = jax.lax.dot_general(
        less.astype(jnp.bfloat16), onehot_b, _DN,
        preferred_element_type=jnp.float32)
    sum_l = jax.lax.dot_general(jnp.where(less, dist, 0.0), onehot, _DN,
                                precision=_HI)
    neg_mean = (sum_l + (kf - cnt_l) * v_pair) / kf
    neg_mean = jnp.where(counts_row < kf, BIG, neg_mean)

    def pos_step(_, carry):
        lo, hi = carry
        mid = 0.5 * (lo + hi)
        cmp = jnp.where(dist >= -mid, onehot, 0.0)
        cnt = jnp.sum(cmp, axis=1, keepdims=True)
        pred = cnt >= kf
        return (jnp.where(pred, lo, mid), jnp.where(pred, mid, hi))

    lo0p = jnp.full((CP, 1), -1.0, dtype=jnp.float32) * (dmax + 2.0)
    hi0p = jnp.zeros((CP, 1), dtype=jnp.float32)
    _, v_p = jax.lax.fori_loop(0, NITER, pos_step, (lo0p, hi0p))

    cmp_p = jnp.where(-dist < v_p, onehot, 0.0)
    cnt_lp = jnp.sum(cmp_p, axis=1, keepdims=True)
    sum_lp = jnp.sum(cmp_p * (-dist), axis=1, keepdims=True)
    pos_mean = -((sum_lp + (kf - cnt_lp) * v_p) / kf)
    pos_mean = jnp.where(counts < kf, -BIG, pos_mean)

    terms = jnp.maximum(ALPHA + pos_mean - neg_mean, 0.0)
    present = counts > 0.0
    present_row = counts_row > 0.0
    diag = (jax.lax.broadcasted_iota(jnp.int32, (CP, CP), 0)
            == jax.lax.broadcasted_iota(jnp.int32, (CP, CP), 1))
    mask = present & present_row & (~diag)
    loss = jnp.sum(jnp.where(mask, terms, 0.0))
    c_n = jnp.sum(present.astype(jnp.float32))
    loss = loss / ((c_n + 1.0) * (c_n * 0.5))
    out_ref[...] = jnp.broadcast_to(loss, (1, 1))


def kernel(x, y):
    xt = x.T
    y32 = y.astype(jnp.int32)
    yrow = y32.reshape(1, N)
    y2d = y32.reshape(N // 128, 128)
    zeros = jnp.zeros((CP, DIM), dtype=jnp.float32)
    part = _sc_center_sums(x, y2d, zeros)
    out = pl.pallas_call(
        _loss_body,
        out_shape=jax.ShapeDtypeStruct((1, 1), jnp.float32),
        compiler_params=pltpu.CompilerParams(
            vmem_limit_bytes=64 * 1024 * 1024),
    )(xt, yrow, part.reshape(2 * CP, DIM))
    return out.reshape(1)


# speedup vs baseline: 172.8940x; 1.3697x over previous
import functools

import jax
import jax.numpy as jnp
from jax.experimental import pallas as pl
from jax.experimental.pallas import tpu as pltpu
from jax.experimental.pallas import tpu_sc as plsc

N = 16384
DIM = 128
NUM_CLASSES = 100
CP = 104
K = 64
ALPHA = 1.0
EPS = 1e-6
NITER = 32
BIG = 1e30

_HI = jax.lax.Precision.HIGHEST
_DN = (((1,), (1,)), ((), ()))


@functools.lru_cache(maxsize=1)
def _sc_center_sums_fn():
    mesh = plsc.VectorSubcoreMesh(core_axis_name="c", subcore_axis_name="s")

    @functools.partial(
        pl.kernel,
        mesh=mesh,
        out_type=jax.ShapeDtypeStruct((2, CP, DIM), jnp.float32),
        scratch_types=[
            pltpu.VMEM((128, DIM), jnp.float32),
            pltpu.VMEM((4, 128), jnp.int32),
            pltpu.VMEM_SHARED((CP, DIM), jnp.float32),
        ],
    )
    def _sc_center_sums(x_hbm, y_hbm, z_hbm, out_hbm, xbuf, ybuf, acc):
        cid = jax.lax.axis_index("c")
        sid = jax.lax.axis_index("s")

        @pl.when(sid == 0)
        def _():
            pltpu.sync_copy(z_hbm, acc)
        plsc.subcore_barrier()

        base = cid * 64 + sid * 4
        pltpu.sync_copy(y_hbm.at[pl.ds(base, 4)], ybuf)
        for g in range(4):
            pltpu.sync_copy(x_hbm.at[pl.ds((base + g) * 128, 128)], xbuf)
            pltpu.sync_copy(xbuf, acc.at[ybuf.at[g]], add=True)
        plsc.subcore_barrier()

        @pl.when(sid == 0)
        def _():
            pltpu.sync_copy(acc, out_hbm.at[cid])

    return _sc_center_sums


def _loss_body(xt_ref, yrow_ref, part_ref, out_ref):
    xt = xt_ref[...]
    yrow = yrow_ref[...]

    member = (jax.lax.broadcasted_iota(jnp.int32, (CP, N), 0) == yrow)
    onehot = member.astype(jnp.float32)
    onehot_b = member.astype(jnp.bfloat16)

    counts = jnp.sum(onehot, axis=1, keepdims=True)
    ones_row = jnp.ones((1, N), dtype=jnp.bfloat16)
    counts_row = jax.lax.dot_general(
        ones_row, onehot_b, _DN,
        preferred_element_type=jnp.float32)

    part = part_ref[...]
    sums = part[0:CP, :] + part[CP:2 * CP, :]
    cpm = sums / jnp.maximum(counts, 1.0) + EPS

    g = jax.lax.dot(cpm, xt, precision=_HI)
    cpsq = jnp.sum(cpm * cpm, axis=1, keepdims=True)
    xsq = jnp.sum(xt * xt, axis=0, keepdims=True)
    dist = jnp.sqrt(jnp.maximum(cpsq - 2.0 * g + xsq, 0.0))

    dmax = jnp.max(dist)
    kf = jnp.float32(K)

    diag = (jax.lax.broadcasted_iota(jnp.int32, (CP, CP), 0)
            == jax.lax.broadcasted_iota(jnp.int32, (CP, CP), 1))

    kmat = jnp.where(diag, counts - (kf - 1.0), kf)

    def neg_step(_, carry):
        lo, hi = carry
        mid = 0.5 * (lo + hi)
        t_s = jax.lax.dot(mid, onehot, precision=_HI)
        cmp = (dist <= t_s).astype(jnp.bfloat16)
        cnt = jax.lax.dot_general(
            cmp, onehot_b, _DN,
            preferred_element_type=jnp.float32)
        pred = cnt >= kmat
        return (jnp.where(pred, lo, mid), jnp.where(pred, mid, hi))

    lo0 = jnp.full((CP, CP), -1.0, dtype=jnp.float32)
    hi0 = jnp.full((CP, CP), 1.0, dtype=jnp.float32) * (dmax + 1.0)
    _, v_pair = jax.lax.fori_loop(0, NITER, neg_step, (lo0, hi0))

    v_s = jax.lax.dot(v_pair, onehot, precision=_HI)
    less = dist < v_s
    cnt_l = jax.lax.dot_general(
        less.astype(jnp.bfloat16), onehot_b, _DN,
        preferred_element_type=jnp.float32)
    sum_l = jax.lax.dot_general(jnp.where(less, dist, 0.0), onehot, _DN,
                                precision=_HI)
    neg_mean = (sum_l + (kf - cnt_l) * v_pair) / kf
    neg_mean = jnp.where(counts_row < kf, BIG, neg_mean)

    s_total = jnp.sum(onehot * dist, axis=1, keepdims=True)
    v_d = jnp.sum(jnp.where(diag, v_pair, 0.0), axis=1, keepdims=True)
    cnt_d = jnp.sum(jnp.where(diag, cnt_l, 0.0), axis=1, keepdims=True)
    sum_d = jnp.sum(jnp.where(diag, sum_l, 0.0), axis=1, keepdims=True)
    pos_mean = (s_total - sum_d + (kf - counts + cnt_d) * v_d) / kf
    pos_mean = jnp.where(counts < kf, -BIG, pos_mean)

    terms = jnp.maximum(ALPHA + pos_mean - neg_mean, 0.0)
    present = counts > 0.0
    present_row = counts_row > 0.0
    mask = present & present_row & (~diag)
    loss = jnp.sum(jnp.where(mask, terms, 0.0))
    c_n = jnp.sum(present.astype(jnp.float32))
    loss = loss / ((c_n + 1.0) * (c_n * 0.5))
    out_ref[...] = jnp.broadcast_to(loss, (1, 1))


def kernel(x, y):
    xt = x.T
    y32 = y.astype(jnp.int32)
    yrow = y32.reshape(1, N)
    y2d = y32.reshape(N // 128, 128)
    zeros = jnp.zeros((CP, DIM), dtype=jnp.float32)
    part = _sc_center_sums_fn()(x, y2d, zeros)
    out = pl.pallas_call(
        _loss_body,
        out_shape=jax.ShapeDtypeStruct((1, 1), jnp.float32),
        compiler_params=pltpu.CompilerParams(
            vmem_limit_bytes=64 * 1024 * 1024),
    )(xt, yrow, part.reshape(2 * CP, DIM))
    return out.reshape(1)


# speedup vs baseline: 187.0800x; 1.0821x over previous
import functools

import jax
import jax.numpy as jnp
from jax.experimental import pallas as pl
from jax.experimental.pallas import tpu as pltpu
from jax.experimental.pallas import tpu_sc as plsc

N = 16384
DIM = 128
NUM_CLASSES = 100
CP = 104
K = 64
ALPHA = 1.0
EPS = 1e-6
NITER = 32
BIG = 1e30

_HI = jax.lax.Precision.HIGHEST
_DN = (((1,), (1,)), ((), ()))


@functools.lru_cache(maxsize=1)
def _sc_center_sums_fn():
    mesh = plsc.VectorSubcoreMesh(core_axis_name="c", subcore_axis_name="s")

    @functools.partial(
        pl.kernel,
        mesh=mesh,
        out_type=jax.ShapeDtypeStruct((2, CP, DIM), jnp.float32),
        scratch_types=[
            pltpu.VMEM((128, DIM), jnp.float32),
            pltpu.VMEM((4, 128), jnp.int32),
            pltpu.VMEM_SHARED((CP, DIM), jnp.float32),
        ],
    )
    def _sc_center_sums(x_hbm, y_hbm, z_hbm, out_hbm, xbuf, ybuf, acc):
        cid = jax.lax.axis_index("c")
        sid = jax.lax.axis_index("s")

        @pl.when(sid == 0)
        def _():
            pltpu.sync_copy(z_hbm, acc)
        plsc.subcore_barrier()

        base = cid * 64 + sid * 4
        pltpu.sync_copy(y_hbm.at[pl.ds(base, 4)], ybuf)
        for g in range(4):
            pltpu.sync_copy(x_hbm.at[pl.ds((base + g) * 128, 128)], xbuf)
            pltpu.sync_copy(xbuf, acc.at[ybuf.at[g]], add=True)
        plsc.subcore_barrier()

        @pl.when(sid == 0)
        def _():
            pltpu.sync_copy(acc, out_hbm.at[cid])

    return _sc_center_sums


def _loss_body(xt_ref, yrow_ref, part_ref, out_ref):
    xt = xt_ref[...]
    yrow = yrow_ref[...]

    member = (jax.lax.broadcasted_iota(jnp.int32, (CP, N), 0) == yrow)
    onehot = member.astype(jnp.float32)
    onehot_b = member.astype(jnp.bfloat16)

    counts = jnp.sum(onehot, axis=1, keepdims=True)
    ones_row = jnp.ones((1, N), dtype=jnp.bfloat16)
    counts_row = jax.lax.dot_general(
        ones_row, onehot_b, _DN,
        preferred_element_type=jnp.float32)

    part = part_ref[...]
    sums = part[0:CP, :] + part[CP:2 * CP, :]
    cpm = sums / jnp.maximum(counts, 1.0) + EPS

    g = jax.lax.dot(cpm, xt, precision=_HI)
    cpsq = jnp.sum(cpm * cpm, axis=1, keepdims=True)
    xsq = jnp.sum(xt * xt, axis=0, keepdims=True)
    dist = jnp.sqrt(jnp.maximum(cpsq - 2.0 * g + xsq, 0.0))

    dmax = jnp.max(dist)
    kf = jnp.float32(K)

    diag = (jax.lax.broadcasted_iota(jnp.int32, (CP, CP), 0)
            == jax.lax.broadcasted_iota(jnp.int32, (CP, CP), 1))

    kmat = jnp.where(diag, counts - (kf - 1.0), kf)

    def neg_step(_, carry):
        lo, hi = carry
        mid = 0.5 * (lo + hi)
        cmp = (dist <= mid).astype(jnp.bfloat16)
        cnt = jax.lax.dot_general(
            cmp, onehot_b, _DN,
            preferred_element_type=jnp.float32)
        dec = (cnt >= kmat).astype(jnp.bfloat16)
        dec_s = jax.lax.dot(dec, onehot_b,
                            preferred_element_type=jnp.float32)
        pred = dec_s > 0.5
        return (jnp.where(pred, lo, mid), jnp.where(pred, mid, hi))

    lo0 = jnp.full((CP, N), -1.0, dtype=jnp.float32)
    hi0 = jnp.full((CP, N), 1.0, dtype=jnp.float32) * (dmax + 1.0)
    _, v_smp = jax.lax.fori_loop(0, NITER, neg_step, (lo0, hi0))

    less = dist < v_smp
    cnt_l = jax.lax.dot_general(
        less.astype(jnp.bfloat16), onehot_b, _DN,
        preferred_element_type=jnp.float32)
    sum_l = jax.lax.dot_general(jnp.where(less, dist, 0.0), onehot, _DN,
                                precision=_HI)
    v_pair = jax.lax.dot_general(v_smp, onehot, _DN,
                                 precision=_HI)
    v_pair = v_pair / jnp.maximum(counts_row, 1.0)
    neg_mean = (sum_l + (kf - cnt_l) * v_pair) / kf
    neg_mean = jnp.where(counts_row < kf, BIG, neg_mean)

    s_total = jnp.sum(onehot * dist, axis=1, keepdims=True)
    v_d = jnp.sum(jnp.where(diag, v_pair, 0.0), axis=1, keepdims=True)
    cnt_d = jnp.sum(jnp.where(diag, cnt_l, 0.0), axis=1, keepdims=True)
    sum_d = jnp.sum(jnp.where(diag, sum_l, 0.0), axis=1, keepdims=True)
    pos_mean = (s_total - sum_d + (kf - counts + cnt_d) * v_d) / kf
    pos_mean = jnp.where(counts < kf, -BIG, pos_mean)

    terms = jnp.maximum(ALPHA + pos_mean - neg_mean, 0.0)
    present = counts > 0.0
    present_row = counts_row > 0.0
    mask = present & present_row & (~diag)
    loss = jnp.sum(jnp.where(mask, terms, 0.0))
    c_n = jnp.sum(present.astype(jnp.float32))
    loss = loss / ((c_n + 1.0) * (c_n * 0.5))
    out_ref[...] = jnp.broadcast_to(loss, (1, 1))


def kernel(x, y):
    xt = x.T
    y32 = y.astype(jnp.int32)
    yrow = y32.reshape(1, N)
    y2d = y32.reshape(N // 128, 128)
    zeros = jnp.zeros((CP, DIM), dtype=jnp.float32)
    part = _sc_center_sums_fn()(x, y2d, zeros)
    out = pl.pallas_call(
        _loss_body,
        out_shape=jax.ShapeDtypeStruct((1, 1), jnp.float32),
        compiler_params=pltpu.CompilerParams(
            vmem_limit_bytes=64 * 1024 * 1024),
    )(xt, yrow, part.reshape(2 * CP, DIM))
    return out.reshape(1)


# speedup vs baseline: 194.6808x; 1.0406x over previous
import functools

import jax
import jax.numpy as jnp
from jax.experimental import pallas as pl
from jax.experimental.pallas import tpu as pltpu
from jax.experimental.pallas import tpu_sc as plsc

N = 16384
DIM = 128
NUM_CLASSES = 100
CP = 104
K = 64
ALPHA = 1.0
EPS = 1e-6
NITER = 32
BIG = 1e30

_HI = jax.lax.Precision.HIGHEST
_DN = (((1,), (1,)), ((), ()))


@functools.lru_cache(maxsize=1)
def _sc_center_sums_fn():
    mesh = plsc.VectorSubcoreMesh(core_axis_name="c", subcore_axis_name="s")

    @functools.partial(
        pl.kernel,
        mesh=mesh,
        out_type=jax.ShapeDtypeStruct((2, CP, DIM), jnp.float32),
        scratch_types=[
            pltpu.VMEM((128, DIM), jnp.float32),
            pltpu.VMEM((4, 128), jnp.int32),
            pltpu.VMEM_SHARED((CP, DIM), jnp.float32),
        ],
    )
    def _sc_center_sums(x_hbm, y_hbm, z_hbm, out_hbm, xbuf, ybuf, acc):
        cid = jax.lax.axis_index("c")
        sid = jax.lax.axis_index("s")

        @pl.when(sid == 0)
        def _():
            pltpu.sync_copy(z_hbm, acc)
        plsc.subcore_barrier()

        base = cid * 64 + sid * 4
        pltpu.sync_copy(y_hbm.at[pl.ds(base, 4)], ybuf)
        for g in range(4):
            pltpu.sync_copy(x_hbm.at[pl.ds((base + g) * 128, 128)], xbuf)
            pltpu.sync_copy(xbuf, acc.at[ybuf.at[g]], add=True)
        plsc.subcore_barrier()

        @pl.when(sid == 0)
        def _():
            pltpu.sync_copy(acc, out_hbm.at[cid])

    return _sc_center_sums


def _loss_body(xt_ref, yrow_ref, part_ref, out_ref):
    xt = xt_ref[...]
    yrow = yrow_ref[...]

    member = (jax.lax.broadcasted_iota(jnp.int32, (CP, N), 0) == yrow)
    onehot = member.astype(jnp.float32)
    onehot_b = member.astype(jnp.bfloat16)

    counts = jnp.sum(onehot, axis=1, keepdims=True)
    ones_row = jnp.ones((1, N), dtype=jnp.bfloat16)
    counts_row = jax.lax.dot_general(
        ones_row, onehot_b, _DN,
        preferred_element_type=jnp.float32)

    part = part_ref[...]
    sums = part[0:CP, :] + part[CP:2 * CP, :]
    cpm = sums / jnp.maximum(counts, 1.0) + EPS

    g = jax.lax.dot(cpm, xt)
    cpsq = jnp.sum(cpm * cpm, axis=1, keepdims=True)
    xsq = jnp.sum(xt * xt, axis=0, keepdims=True)
    dist = jnp.sqrt(jnp.maximum(cpsq - 2.0 * g + xsq, 0.0))

    dmax = jnp.max(dist)
    kf = jnp.float32(K)

    diag = (jax.lax.broadcasted_iota(jnp.int32, (CP, CP), 0)
            == jax.lax.broadcasted_iota(jnp.int32, (CP, CP), 1))

    kmat = jnp.where(diag, counts - (kf - 1.0), kf)

    def neg_step(_, carry):
        lo, hi = carry
        mid = 0.5 * (lo + hi)
        cmp = (dist <= mid).astype(jnp.bfloat16)
        cnt = jax.lax.dot_general(
            cmp, onehot_b, _DN,
            preferred_element_type=jnp.float32)
        dec = (cnt >= kmat).astype(jnp.bfloat16)
        dec_s = jax.lax.dot(dec, onehot_b,
                            preferred_element_type=jnp.float32)
        pred = dec_s > 0.5
        return (jnp.where(pred, lo, mid), jnp.where(pred, mid, hi))

    lo0 = jnp.full((CP, N), -1.0, dtype=jnp.float32)
    hi0 = jnp.full((CP, N), 1.0, dtype=jnp.float32) * (dmax + 1.0)
    _, v_smp = jax.lax.fori_loop(0, NITER, neg_step, (lo0, hi0))

    less = dist < v_smp
    cnt_l = jax.lax.dot_general(
        less.astype(jnp.bfloat16), onehot_b, _DN,
        preferred_element_type=jnp.float32)

    def _split_sum(vals):
        v_hi = vals.astype(jnp.bfloat16)
        v_lo = (vals - v_hi.astype(jnp.float32)).astype(jnp.bfloat16)
        s_hi = jax.lax.dot_general(v_hi, onehot_b, _DN,
                                   preferred_element_type=jnp.float32)
        s_lo = jax.lax.dot_general(v_lo, onehot_b, _DN,
                                   preferred_element_type=jnp.float32)
        return s_hi + s_lo

    sum_l = _split_sum(jnp.where(less, dist, 0.0))
    v_pair = _split_sum(v_smp) / jnp.maximum(counts_row, 1.0)
    neg_mean = (sum_l + (kf - cnt_l) * v_pair) / kf
    neg_mean = jnp.where(counts_row < kf, BIG, neg_mean)

    s_total = jnp.sum(onehot * dist, axis=1, keepdims=True)
    v_d = jnp.sum(jnp.where(diag, v_pair, 0.0), axis=1, keepdims=True)
    cnt_d = jnp.sum(jnp.where(diag, cnt_l, 0.0), axis=1, keepdims=True)
    sum_d = jnp.sum(jnp.where(diag, sum_l, 0.0), axis=1, keepdims=True)
    pos_mean = (s_total - sum_d + (kf - counts + cnt_d) * v_d) / kf
    pos_mean = jnp.where(counts < kf, -BIG, pos_mean)

    terms = jnp.maximum(ALPHA + pos_mean - neg_mean, 0.0)
    present = counts > 0.0
    present_row = counts_row > 0.0
    mask = present & present_row & (~diag)
    loss = jnp.sum(jnp.where(mask, terms, 0.0))
    c_n = jnp.sum(present.astype(jnp.float32))
    loss = loss / ((c_n + 1.0) * (c_n * 0.5))
    out_ref[...] = jnp.broadcast_to(loss, (1, 1))


def kernel(x, y):
    xt = x.T
    y32 = y.astype(jnp.int32)
    yrow = y32.reshape(1, N)
    y2d = y32.reshape(N // 128, 128)
    zeros = jnp.zeros((CP, DIM), dtype=jnp.float32)
    part = _sc_center_sums_fn()(x, y2d, zeros)
    out = pl.pallas_call(
        _loss_body,
        out_shape=jax.ShapeDtypeStruct((1, 1), jnp.float32),
        compiler_params=pltpu.CompilerParams(
            vmem_limit_bytes=64 * 1024 * 1024),
    )(xt, yrow, part.reshape(2 * CP, DIM))
    return out.reshape(1)
